# baseline (speedup 1.0000x reference)
import functools

import jax
import jax.numpy as jnp
from jax import lax
from jax.experimental import pallas as pl
from jax.experimental.pallas import tpu as pltpu

EPS = 1e-5
_VMEM_LIMIT = 56 * 1024 * 1024


def _round_up(x, m):
    return (x + m - 1) // m * m


def _fold(s, q, g, b, count, eps):
    mean = s * (1.0 / count)
    var = jnp.maximum(q * (1.0 / count) - mean * mean, 0.0)
    inv = lax.rsqrt(var + eps)
    scale = g * inv
    shift = b - mean * scale
    return scale, shift


def _mega_body(x_ref, w1_ref, w2_ref, w3_ref, g1_ref, b1_ref, g2_ref, b2_ref,
               g3_ref, b3_ref, ml_ref, mr_ref, o_ref,
               xb_s, y1_s, y2_s, s1_s, q1_s, s2_s, q2_s, cs_s, gram_s,
               *, gs_a, n_img, n_c, gs_d, tm, tc, hw, width, pad_rows, count):
    i = pl.program_id(0)

    @pl.when(i == 0)
    def _():
        s1_s[...] = jnp.zeros_like(s1_s)
        q1_s[...] = jnp.zeros_like(q1_s)
        s2_s[...] = jnp.zeros_like(s2_s)
        q2_s[...] = jnp.zeros_like(q2_s)
        cs_s[...] = jnp.zeros_like(cs_s)
        gram_s[...] = jnp.zeros_like(gram_s)

    @pl.when(i < gs_a)
    def _():
        x = x_ref[...]
        xb = x.astype(jnp.bfloat16)
        row = pl.multiple_of(i * tm, tm)
        xb_s[pl.ds(row, tm), :] = xb
        y = jnp.dot(xb, w1_ref[...], preferred_element_type=jnp.float32)
        y1_s[pl.ds(row, tm), :] = y.astype(jnp.bfloat16)
        s1_s[...] += jnp.sum(y, axis=0, keepdims=True)
        q1_s[...] += jnp.sum(y * y, axis=0, keepdims=True)

    @pl.when((i >= gs_a) & (i < gs_a + n_img))
    def _():
        img = i - gs_a
        scale, shift = _fold(s1_s[...], q1_s[...], g1_ref[...], b1_ref[...],
                             count, EPS)
        row = pl.multiple_of(img * hw, hw)
        yb = y1_s[pl.ds(row, hw), :]
        a = jnp.maximum(yb.astype(jnp.float32) * scale + shift, 0.0)
        ab = a.astype(jnp.bfloat16)
        cp = ab.shape[1]

        zpad = jnp.zeros((pad_rows, cp), jnp.bfloat16)
        ap = jnp.concatenate([zpad, ab, zpad], axis=0)
        lp = hw + 2 * pad_rows
        zrow = jnp.zeros((1, cp), jnp.bfloat16)
        a_l = jnp.concatenate([zrow, ap[:lp - 1]], axis=0) * ml_ref[...]
        a_r = jnp.concatenate([ap[1:], zrow], axis=0) * mr_ref[...]
        p3 = jnp.concatenate([a_l, ap, a_r], axis=1)

        c_all = jnp.dot(p3, w2_ref[...], preferred_element_type=jnp.float32)
        cout = w2_ref.shape[1] // 3
        acc = (c_all[pad_rows - width: pad_rows - width + hw, 0:cout]
               + c_all[pad_rows: pad_rows + hw, cout:2 * cout]
               + c_all[pad_rows + width: pad_rows + width + hw,
                       2 * cout:3 * cout])

        y2_s[pl.ds(row, hw), :] = acc.astype(jnp.bfloat16)
        s2_s[...] += jnp.sum(acc, axis=0, keepdims=True)
        q2_s[...] += jnp.sum(acc * acc, axis=0, keepdims=True)

    @pl.when((i >= gs_a + n_img) & (i < gs_a + n_img + n_c))
    def _():
        j = i - gs_a - n_img
        scale, shift = _fold(s2_s[...], q2_s[...], g2_ref[...], b2_ref[...],
                             count, EPS)
        row = pl.multiple_of(j * tc, tc)
        yb = y2_s[pl.ds(row, tc), :]
        a2 = jnp.maximum(yb.astype(jnp.float32) * scale + shift, 0.0)
        ab = a2.astype(jnp.bfloat16)
        cs_s[...] += jnp.sum(a2, axis=0, keepdims=True)
        gram_s[...] += lax.dot_general(ab, ab, (((0,), (0,)), ((), ())),
                                       preferred_element_type=jnp.float32)

    @pl.when(i >= gs_a + n_img + n_c)
    def _():
        j = i - gs_a - n_img - n_c
        sc2, sh2 = _fold(s2_s[...], q2_s[...], g2_ref[...], b2_ref[...],
                         count, EPS)
        row = pl.multiple_of(j * tm, tm)
        yb = y2_s[pl.ds(row, tm), :]
        a2 = jnp.maximum(yb.astype(jnp.float32) * sc2 + sh2, 0.0)
        ab = a2.astype(jnp.bfloat16)

        w3 = w3_ref[...]
        s3 = jnp.dot(cs_s[...], w3, preferred_element_type=jnp.float32)
        gw = jnp.dot(gram_s[...], w3, preferred_element_type=jnp.float32)
        q3 = jnp.sum(w3 * gw, axis=0, keepdims=True)
        sc3, sh3 = _fold(s3, q3, g3_ref[...], b3_ref[...], count, EPS)

        y3 = jnp.dot(ab, w3.astype(jnp.bfloat16),
                     preferred_element_type=jnp.float32)
        xres = xb_s[pl.ds(row, tm), :].astype(jnp.float32)
        o_ref[...] = jnp.maximum(y3 * sc3 + sh3 + xres, 0.0)


def _mega_packed_body(x_ref, w1_ref, w2_ref, w3a_ref, w3b_ref, w3p_ref,
                      g1_ref, b1_ref, g2_ref, b2_ref, g3_ref, b3_ref,
                      ml_ref, mr_ref, bd_ref, o_ref,
                      xb_s, y1_s, y2_s, s1_s, q1_s, s2_s, q2_s, cs_s, gram_s,
                      *, gs_a, n_pair, n_c, tc, hw, width, pad_rows, count,
                      pr):
    i = pl.program_id(0)

    @pl.when(i == 0)
    def _():
        s1_s[...] = jnp.zeros_like(s1_s)
        q1_s[...] = jnp.zeros_like(q1_s)
        s2_s[...] = jnp.zeros_like(s2_s)
        q2_s[...] = jnp.zeros_like(q2_s)
        cs_s[...] = jnp.zeros_like(cs_s)
        gram_s[...] = jnp.zeros_like(gram_s)

    def tot(s):
        return s + jnp.concatenate([s[:, pr:], s[:, :pr]], axis=1)

    @pl.when(i < gs_a)
    def _():
        x = x_ref[...]
        xb = x.astype(jnp.bfloat16)
        row = pl.multiple_of(i * hw, hw)
        xb_s[pl.ds(row, hw), :] = xb
        y = jnp.dot(xb, w1_ref[...], preferred_element_type=jnp.float32)
        prow = pl.multiple_of((i // 2) * hw, hw)
        yb = y.astype(jnp.bfloat16)

        @pl.when(i % 2 == 0)
        def _():
            y1_s[pl.ds(prow, hw), 0:pr] = yb[:, 0:pr]

        @pl.when(i % 2 == 1)
        def _():
            y1_s[pl.ds(prow, hw), pr:2 * pr] = yb[:, 0:pr]

        s1_s[...] += jnp.sum(y, axis=0, keepdims=True)
        q1_s[...] += jnp.sum(y * y, axis=0, keepdims=True)

    @pl.when((i >= gs_a) & (i < gs_a + n_pair))
    def _():
        p = i - gs_a
        scale, shift = _fold(tot(s1_s[...]), tot(q1_s[...]),
                             g1_ref[...], b1_ref[...], count, EPS)
        row = pl.multiple_of(p * hw, hw)
        ypk = y1_s[pl.ds(row, hw), :]
        a = jnp.maximum(ypk.astype(jnp.float32) * scale + shift, 0.0)
        ab = a.astype(jnp.bfloat16)
        cp = ab.shape[1]

        zpad = jnp.zeros((pad_rows, cp), jnp.bfloat16)
        ap = jnp.concatenate([zpad, ab, zpad], axis=0)
        lp = hw + 2 * pad_rows
        zrow = jnp.zeros((1, cp), jnp.bfloat16)
        a_l = jnp.concatenate([zrow, ap[:lp - 1]], axis=0) * ml_ref[...]
        a_r = jnp.concatenate([ap[1:], zrow], axis=0) * mr_ref[...]
        p3 = jnp.concatenate([a_l, ap, a_r], axis=1)

        c_all = jnp.dot(p3, w2_ref[...], preferred_element_type=jnp.float32)
        acc = (c_all[pad_rows - width: pad_rows - width + hw, 0:cp]
               + c_all[pad_rows: pad_rows + hw, cp:2 * cp]
               + c_all[pad_rows + width: pad_rows + width + hw, 2 * cp:3 * cp])

        y2_s[pl.ds(row, hw), :] = acc.astype(jnp.bfloat16)
        s2_s[...] += jnp.sum(acc, axis=0, keepdims=True)
        q2_s[...] += jnp.sum(acc * acc, axis=0, keepdims=True)

    @pl.when((i >= gs_a + n_pair) & (i < gs_a + n_pair + n_c))
    def _():
        j = i - gs_a - n_pair
        scale, shift = _fold(tot(s2_s[...]), tot(q2_s[...]),
                             g2_ref[...], b2_ref[...], count, EPS)
        row = pl.multiple_of(j * tc, tc)
        ypk = y2_s[pl.ds(row, tc), :]
        a2 = jnp.maximum(ypk.astype(jnp.float32) * scale + shift, 0.0)
        ab = a2.astype(jnp.bfloat16)
        cs_s[...] += jnp.sum(a2, axis=0, keepdims=True)
        gram_s[...] += lax.dot_general(ab, ab, (((0,), (0,)), ((), ())),
                                       preferred_element_type=jnp.float32)

    @pl.when(i >= gs_a + n_pair + n_c)
    def _():
        j = i - gs_a - n_pair - n_c
        sc2, sh2 = _fold(tot(s2_s[...]), tot(q2_s[...]),
                         g2_ref[...], b2_ref[...], count, EPS)
        row = pl.multiple_of(j * hw, hw)
        ypk = y2_s[pl.ds(row, hw), :]
        a2 = jnp.maximum(ypk.astype(jnp.float32) * sc2 + sh2, 0.0)
        ab = a2.astype(jnp.bfloat16)

        w3p = w3p_ref[...]
        gp = gram_s[...] * bd_ref[...]
        s3 = jnp.dot(cs_s[...], w3p, preferred_element_type=jnp.float32)
        gw = jnp.dot(gp, w3p, preferred_element_type=jnp.float32)
        q3 = jnp.sum(w3p * gw, axis=0, keepdims=True)
        sc3, sh3 = _fold(s3, q3, g3_ref[...], b3_ref[...], count, EPS)

        y3a = jnp.dot(ab, w3a_ref[...], preferred_element_type=jnp.float32)
        y3b = jnp.dot(ab, w3b_ref[...], preferred_element_type=jnp.float32)
        xrow = pl.multiple_of(j * 2 * hw, hw)
        xx = xb_s[pl.ds(xrow, 2 * hw), :].astype(jnp.float32)
        o_ref[0:hw, :] = jnp.maximum(y3a * sc3 + sh3 + xx[0:hw], 0.0)
        o_ref[hw:2 * hw, :] = jnp.maximum(y3b * sc3 + sh3 + xx[hw:2 * hw], 0.0)


def _kernel_packed(x2d, w1, w2, w3, g1, b1, g2, b2, g3, b3,
                   n, h, w, cin_pad):
    hw = h * w
    m = n * hw
    m2 = m // 2
    cp = w2.shape[1]
    c4 = w3.shape[1]
    pr = cp // 2
    count = float(m)

    gs_a = n
    n_pair = n // 2
    n_c = 2 if (m2 // 2) % 16 == 0 else 1
    tc = m2 // n_c
    pad_rows = _round_up(w + 1, 16)
    lp = hw + 2 * pad_rows
    grid = gs_a + n_pair + n_c + n_pair

    w1b = w1.astype(jnp.bfloat16)
    w2r = w2[:, :pr, :pr]
    zz = jnp.zeros_like(w2r)
    w2bd = jnp.concatenate([jnp.concatenate([w2r, zz], axis=2),
                            jnp.concatenate([zz, w2r], axis=2)], axis=1)
    w2p = (w2bd.astype(jnp.bfloat16).reshape(3, 3, cp, cp)
           .transpose(1, 2, 0, 3).reshape(3 * cp, 3 * cp))
    w3a = w3.astype(jnp.bfloat16)
    w3b = jnp.concatenate([jnp.zeros((pr, c4), w3.dtype), w3[:pr]],
                          axis=0).astype(jnp.bfloat16)
    w3p = jnp.concatenate([w3[:pr], w3[:pr]], axis=0)

    g1p = jnp.concatenate([g1[:, :pr], g1[:, :pr]], axis=1)
    b1p = jnp.concatenate([b1[:, :pr], b1[:, :pr]], axis=1)
    g2p = jnp.concatenate([g2[:, :pr], g2[:, :pr]], axis=1)
    b2p = jnp.concatenate([b2[:, :pr], b2[:, :pr]], axis=1)

    col = (jnp.arange(lp, dtype=jnp.int32) - pad_rows) % w
    ml = (col >= 1).astype(jnp.bfloat16).reshape(lp, 1)
    mr = (col <= w - 2).astype(jnp.bfloat16).reshape(lp, 1)
    blk = jnp.arange(cp) // pr
    bd = (blk[:, None] == blk[None, :]).astype(jnp.float32)

    kern = functools.partial(
        _mega_packed_body, gs_a=gs_a, n_pair=n_pair, n_c=n_c, tc=tc,
        hw=hw, width=w, pad_rows=pad_rows, count=count, pr=pr)

    def _x_map(i):
        return (jnp.minimum(i, gs_a - 1), 0)

    def _o_map(i):
        return (jnp.maximum(i - (gs_a + n_pair + n_c), 0), 0)

    return pl.pallas_call(
        kern,
        grid=(grid,),
        in_specs=[pl.BlockSpec((hw, cin_pad), _x_map),
                  pl.BlockSpec((cin_pad, cp), lambda i: (0, 0)),
                  pl.BlockSpec((3 * cp, 3 * cp), lambda i: (0, 0)),
                  pl.BlockSpec((cp, c4), lambda i: (0, 0)),
                  pl.BlockSpec((cp, c4), lambda i: (0, 0)),
                  pl.BlockSpec((cp, c4), lambda i: (0, 0)),
                  pl.BlockSpec((1, cp), lambda i: (0, 0)),
                  pl.BlockSpec((1, cp), lambda i: (0, 0)),
                  pl.BlockSpec((1, cp), lambda i: (0, 0)),
                  pl.BlockSpec((1, cp), lambda i: (0, 0)),
                  pl.BlockSpec((1, c4), lambda i: (0, 0)),
                  pl.BlockSpec((1, c4), lambda i: (0, 0)),
                  pl.BlockSpec((lp, 1), lambda i: (0, 0)),
                  pl.BlockSpec((lp, 1), lambda i: (0, 0)),
                  pl.BlockSpec((cp, cp), lambda i: (0, 0))],
        out_specs=pl.BlockSpec((2 * hw, c4), _o_map),
        out_shape=jax.ShapeDtypeStruct((m, c4), jnp.float32),
        scratch_shapes=[pltpu.VMEM((m, cin_pad), jnp.bfloat16),
                        pltpu.VMEM((m2, cp), jnp.bfloat16),
                        pltpu.VMEM((m2, cp), jnp.bfloat16),
                        pltpu.VMEM((1, cp), jnp.float32),
                        pltpu.VMEM((1, cp), jnp.float32),
                        pltpu.VMEM((1, cp), jnp.float32),
                        pltpu.VMEM((1, cp), jnp.float32),
                        pltpu.VMEM((1, cp), jnp.float32),
                        pltpu.VMEM((cp, cp), jnp.float32)],
        compiler_params=pltpu.CompilerParams(
            dimension_semantics=("arbitrary",),
            vmem_limit_bytes=_VMEM_LIMIT),
    )(x2d, w1b, w2p, w3a, w3b, w3p, g1p, b1p, g2p, b2p, g3, b3, ml, mr, bd)


def kernel(x_nhwc, w1, w2, w3, g1, b1, g2, b2, g3, b3):
    n, h, w, cin = x_nhwc.shape
    m = n * h * w
    hw = h * w
    cin_pad = w1.shape[0]

    x2d = x_nhwc.reshape(m, cin)
    if cin_pad != cin:
        x2d = jnp.pad(x2d, ((0, 0), (0, cin_pad - cin)))

    if (n % 2 == 0 and hw % 16 == 0
            and w2.shape[1] == w2.shape[2] == 2 * (cin // 4)):
        out = _kernel_packed(x2d, w1, w2, w3, g1, b1, g2, b2, g3, b3,
                             n, h, w, cin_pad)
        if cin_pad != cin:
            out = out[:, :cin]
        return out.reshape(n, h, w, cin)

    cp = w2.shape[1]
    cout2 = w2.shape[2]
    c4 = w3.shape[1]
    count = float(m)

    tm = hw
    gs_a = m // tm
    n_img = n
    n_c = 2 if (m // 2) % 8 == 0 else 1
    tc = m // n_c
    gs_d = m // tm
    pad_rows = _round_up(w + 1, 16)
    lp = hw + 2 * pad_rows
    grid = gs_a + n_img + n_c + gs_d

    w1b = w1.astype(jnp.bfloat16)
    w2b = (w2.astype(jnp.bfloat16).reshape(3, 3, cp, cout2)
           .transpose(1, 2, 0, 3).reshape(3 * cp, 3 * cout2))

    col = (jnp.arange(lp, dtype=jnp.int32) - pad_rows) % w
    ml = (col >= 1).astype(jnp.bfloat16).reshape(lp, 1)
    mr = (col <= w - 2).astype(jnp.bfloat16).reshape(lp, 1)

    kern = functools.partial(
        _mega_body, gs_a=gs_a, n_img=n_img, n_c=n_c, gs_d=gs_d,
        tm=tm, tc=tc, hw=hw, width=w, pad_rows=pad_rows, count=count)

    def _x_map(i):
        return (jnp.minimum(i, gs_a - 1), 0)

    def _o_map(i):
        return (jnp.maximum(i - (gs_a + n_img + n_c), 0), 0)

    out = pl.pallas_call(
        kern,
        grid=(grid,),
        in_specs=[pl.BlockSpec((tm, cin_pad), _x_map),
                  pl.BlockSpec((cin_pad, cp), lambda i: (0, 0)),
                  pl.BlockSpec((3 * cp, 3 * cout2), lambda i: (0, 0)),
                  pl.BlockSpec((cp, c4), lambda i: (0, 0)),
                  pl.BlockSpec((1, cp), lambda i: (0, 0)),
                  pl.BlockSpec((1, cp), lambda i: (0, 0)),
                  pl.BlockSpec((1, cp), lambda i: (0, 0)),
                  pl.BlockSpec((1, cp), lambda i: (0, 0)),
                  pl.BlockSpec((1, c4), lambda i: (0, 0)),
                  pl.BlockSpec((1, c4), lambda i: (0, 0)),
                  pl.BlockSpec((lp, 1), lambda i: (0, 0)),
                  pl.BlockSpec((lp, 1), lambda i: (0, 0))],
        out_specs=pl.BlockSpec((tm, c4), _o_map),
        out_shape=jax.ShapeDtypeStruct((m, c4), jnp.float32),
        scratch_shapes=[pltpu.VMEM((m, cin_pad), jnp.bfloat16),
                        pltpu.VMEM((m, cp), jnp.bfloat16),
                        pltpu.VMEM((m, cout2), jnp.bfloat16),
                        pltpu.VMEM((1, cp), jnp.float32),
                        pltpu.VMEM((1, cp), jnp.float32),
                        pltpu.VMEM((1, cout2), jnp.float32),
                        pltpu.VMEM((1, cout2), jnp.float32),
                        pltpu.VMEM((1, cout2), jnp.float32),
                        pltpu.VMEM((cout2, cout2), jnp.float32)],
        compiler_params=pltpu.CompilerParams(
            dimension_semantics=("arbitrary",),
            vmem_limit_bytes=_VMEM_LIMIT),
    )(x2d, w1b, w2b, w3, g1, b1, g2, b2, g3, b3, ml, mr)

    if cin_pad != cin:
        out = out[:, :cin]
    return out.reshape(n, h, w, cin)


# speedup vs baseline: 1.0475x; 1.0475x over previous
import functools

import jax
import jax.numpy as jnp
from jax import lax
from jax.experimental import pallas as pl
from jax.experimental.pallas import tpu as pltpu

EPS = 1e-5
_VMEM_LIMIT = 56 * 1024 * 1024


def _round_up(x, m):
    return (x + m - 1) // m * m


def _fold(s, q, g, b, count, eps):
    mean = s * (1.0 / count)
    var = jnp.maximum(q * (1.0 / count) - mean * mean, 0.0)
    inv = lax.rsqrt(var + eps)
    scale = g * inv
    shift = b - mean * scale
    return scale, shift


def _mega_body(x_ref, w1_ref, w2_ref, w3_ref, g1_ref, b1_ref, g2_ref, b2_ref,
               g3_ref, b3_ref, ml_ref, mr_ref, o_ref,
               xb_s, y1_s, y2_s, s1_s, q1_s, s2_s, q2_s, cs_s, gram_s,
               *, gs_a, n_img, n_c, gs_d, tm, tc, hw, width, pad_rows, count):
    i = pl.program_id(0)

    @pl.when(i == 0)
    def _():
        s1_s[...] = jnp.zeros_like(s1_s)
        q1_s[...] = jnp.zeros_like(q1_s)
        s2_s[...] = jnp.zeros_like(s2_s)
        q2_s[...] = jnp.zeros_like(q2_s)
        cs_s[...] = jnp.zeros_like(cs_s)
        gram_s[...] = jnp.zeros_like(gram_s)

    @pl.when(i < gs_a)
    def _():
        x = x_ref[...]
        xb = x.astype(jnp.bfloat16)
        row = pl.multiple_of(i * tm, tm)
        xb_s[pl.ds(row, tm), :] = xb
        y = jnp.dot(xb, w1_ref[...], preferred_element_type=jnp.float32)
        y1_s[pl.ds(row, tm), :] = y.astype(jnp.bfloat16)
        s1_s[...] += jnp.sum(y, axis=0, keepdims=True)
        q1_s[...] += jnp.sum(y * y, axis=0, keepdims=True)

    @pl.when((i >= gs_a) & (i < gs_a + n_img))
    def _():
        img = i - gs_a
        scale, shift = _fold(s1_s[...], q1_s[...], g1_ref[...], b1_ref[...],
                             count, EPS)
        row = pl.multiple_of(img * hw, hw)
        yb = y1_s[pl.ds(row, hw), :]
        a = jnp.maximum(yb.astype(jnp.float32) * scale + shift, 0.0)
        ab = a.astype(jnp.bfloat16)
        cp = ab.shape[1]

        zpad = jnp.zeros((pad_rows, cp), jnp.bfloat16)
        ap = jnp.concatenate([zpad, ab, zpad], axis=0)
        lp = hw + 2 * pad_rows
        zrow = jnp.zeros((1, cp), jnp.bfloat16)
        a_l = jnp.concatenate([zrow, ap[:lp - 1]], axis=0) * ml_ref[...]
        a_r = jnp.concatenate([ap[1:], zrow], axis=0) * mr_ref[...]
        p3 = jnp.concatenate([a_l, ap, a_r], axis=1)

        c_all = jnp.dot(p3, w2_ref[...], preferred_element_type=jnp.float32)
        cout = w2_ref.shape[1] // 3
        acc = (c_all[pad_rows - width: pad_rows - width + hw, 0:cout]
               + c_all[pad_rows: pad_rows + hw, cout:2 * cout]
               + c_all[pad_rows + width: pad_rows + width + hw,
                       2 * cout:3 * cout])

        y2_s[pl.ds(row, hw), :] = acc.astype(jnp.bfloat16)
        s2_s[...] += jnp.sum(acc, axis=0, keepdims=True)
        q2_s[...] += jnp.sum(acc * acc, axis=0, keepdims=True)

    @pl.when((i >= gs_a + n_img) & (i < gs_a + n_img + n_c))
    def _():
        j = i - gs_a - n_img
        scale, shift = _fold(s2_s[...], q2_s[...], g2_ref[...], b2_ref[...],
                             count, EPS)
        row = pl.multiple_of(j * tc, tc)
        yb = y2_s[pl.ds(row, tc), :]
        a2 = jnp.maximum(yb.astype(jnp.float32) * scale + shift, 0.0)
        ab = a2.astype(jnp.bfloat16)
        cs_s[...] += jnp.sum(a2, axis=0, keepdims=True)
        gram_s[...] += lax.dot_general(ab, ab, (((0,), (0,)), ((), ())),
                                       preferred_element_type=jnp.float32)

    @pl.when(i >= gs_a + n_img + n_c)
    def _():
        j = i - gs_a - n_img - n_c
        sc2, sh2 = _fold(s2_s[...], q2_s[...], g2_ref[...], b2_ref[...],
                         count, EPS)
        row = pl.multiple_of(j * tm, tm)
        yb = y2_s[pl.ds(row, tm), :]
        a2 = jnp.maximum(yb.astype(jnp.float32) * sc2 + sh2, 0.0)
        ab = a2.astype(jnp.bfloat16)

        w3 = w3_ref[...]
        s3 = jnp.dot(cs_s[...], w3, preferred_element_type=jnp.float32)
        gw = jnp.dot(gram_s[...], w3, preferred_element_type=jnp.float32)
        q3 = jnp.sum(w3 * gw, axis=0, keepdims=True)
        sc3, sh3 = _fold(s3, q3, g3_ref[...], b3_ref[...], count, EPS)

        y3 = jnp.dot(ab, w3.astype(jnp.bfloat16),
                     preferred_element_type=jnp.float32)
        xres = xb_s[pl.ds(row, tm), :].astype(jnp.float32)
        o_ref[...] = jnp.maximum(y3 * sc3 + sh3 + xres, 0.0)


def _mega_packed_body(x_ref, w1_ref, w2_ref, w3a_ref, w3b_ref, w3p_ref,
                      g1_ref, b1_ref, g2_ref, b2_ref, g3_ref, b3_ref,
                      ml_ref, mr_ref, bd_ref, o_ref,
                      xb_s, y1_s, y2_s, s1_s, q1_s, s2_s, q2_s, cs_s, gram_s,
                      *, gs_a, n_pair, n_c, tc, hw, width, pad_rows, count,
                      pr):
    i = pl.program_id(0)

    @pl.when(i == 0)
    def _():
        s1_s[...] = jnp.zeros_like(s1_s)
        q1_s[...] = jnp.zeros_like(q1_s)
        s2_s[...] = jnp.zeros_like(s2_s)
        q2_s[...] = jnp.zeros_like(q2_s)
        cs_s[...] = jnp.zeros_like(cs_s)
        gram_s[...] = jnp.zeros_like(gram_s)

    def tot(s):
        return s + jnp.concatenate([s[:, pr:], s[:, :pr]], axis=1)

    @pl.when(i < gs_a)
    def _():
        x = x_ref[...]
        xb = x.astype(jnp.bfloat16)
        row = pl.multiple_of(i * hw, hw)
        xb_s[pl.ds(row, hw), :] = xb
        y = jnp.dot(xb, w1_ref[...], preferred_element_type=jnp.float32)
        y1_s[pl.ds(row, hw), :] = y.astype(jnp.bfloat16)
        s1_s[...] += jnp.sum(y, axis=0, keepdims=True)
        q1_s[...] += jnp.sum(y * y, axis=0, keepdims=True)

    @pl.when((i >= gs_a) & (i < gs_a + n_pair))
    def _():
        p = i - gs_a
        scale, shift = _fold(tot(s1_s[...]), tot(q1_s[...]),
                             g1_ref[...], b1_ref[...], count, EPS)
        row = pl.multiple_of(p * hw, hw)
        rowa = pl.multiple_of(p * 2 * hw, hw)
        rowb = pl.multiple_of(p * 2 * hw + hw, hw)
        ypk = jnp.concatenate([y1_s[pl.ds(rowa, hw), 0:pr],
                               y1_s[pl.ds(rowb, hw), 0:pr]], axis=1)
        a = jnp.maximum(ypk.astype(jnp.float32) * scale + shift, 0.0)
        ab = a.astype(jnp.bfloat16)
        cp = ab.shape[1]

        zpad = jnp.zeros((pad_rows, cp), jnp.bfloat16)
        ap = jnp.concatenate([zpad, ab, zpad], axis=0)
        lp = hw + 2 * pad_rows
        zrow = jnp.zeros((1, cp), jnp.bfloat16)
        a_l = jnp.concatenate([zrow, ap[:lp - 1]], axis=0) * ml_ref[...]
        a_r = jnp.concatenate([ap[1:], zrow], axis=0) * mr_ref[...]
        p3 = jnp.concatenate([a_l, ap, a_r], axis=1)

        c_all = jnp.dot(p3, w2_ref[...], preferred_element_type=jnp.float32)
        acc = (c_all[pad_rows - width: pad_rows - width + hw, 0:cp]
               + c_all[pad_rows: pad_rows + hw, cp:2 * cp]
               + c_all[pad_rows + width: pad_rows + width + hw, 2 * cp:3 * cp])

        y2_s[pl.ds(row, hw), :] = acc.astype(jnp.bfloat16)
        s2_s[...] += jnp.sum(acc, axis=0, keepdims=True)
        q2_s[...] += jnp.sum(acc * acc, axis=0, keepdims=True)

    @pl.when((i >= gs_a + n_pair) & (i < gs_a + n_pair + n_c))
    def _():
        j = i - gs_a - n_pair
        scale, shift = _fold(tot(s2_s[...]), tot(q2_s[...]),
                             g2_ref[...], b2_ref[...], count, EPS)
        row = pl.multiple_of(j * tc, tc)
        ypk = y2_s[pl.ds(row, tc), :]
        a2 = jnp.maximum(ypk.astype(jnp.float32) * scale + shift, 0.0)
        ab = a2.astype(jnp.bfloat16)
        cs_s[...] += jnp.sum(a2, axis=0, keepdims=True)
        gram_s[...] += lax.dot_general(ab, ab, (((0,), (0,)), ((), ())),
                                       preferred_element_type=jnp.float32)

    @pl.when(i >= gs_a + n_pair + n_c)
    def _():
        j = i - gs_a - n_pair - n_c
        sc2, sh2 = _fold(tot(s2_s[...]), tot(q2_s[...]),
                         g2_ref[...], b2_ref[...], count, EPS)
        row = pl.multiple_of(j * hw, hw)
        ypk = y2_s[pl.ds(row, hw), :]
        a2 = jnp.maximum(ypk.astype(jnp.float32) * sc2 + sh2, 0.0)
        ab = a2.astype(jnp.bfloat16)

        w3p = w3p_ref[...]
        gp = gram_s[...] * bd_ref[...]
        s3 = jnp.dot(cs_s[...], w3p, preferred_element_type=jnp.float32)
        gw = jnp.dot(gp, w3p, preferred_element_type=jnp.float32)
        q3 = jnp.sum(w3p * gw, axis=0, keepdims=True)
        sc3, sh3 = _fold(s3, q3, g3_ref[...], b3_ref[...], count, EPS)

        y3a = jnp.dot(ab, w3a_ref[...], preferred_element_type=jnp.float32)
        y3b = jnp.dot(ab, w3b_ref[...], preferred_element_type=jnp.float32)
        xrow = pl.multiple_of(j * 2 * hw, hw)
        xx = xb_s[pl.ds(xrow, 2 * hw), :].astype(jnp.float32)
        o_ref[0:hw, :] = jnp.maximum(y3a * sc3 + sh3 + xx[0:hw], 0.0)
        o_ref[hw:2 * hw, :] = jnp.maximum(y3b * sc3 + sh3 + xx[hw:2 * hw], 0.0)


def _kernel_packed(x2d, w1, w2, w3, g1, b1, g2, b2, g3, b3,
                   n, h, w, cin_pad):
    hw = h * w
    m = n * hw
    m2 = m // 2
    cp = w2.shape[1]
    c4 = w3.shape[1]
    pr = cp // 2
    count = float(m)

    gs_a = n
    n_pair = n // 2
    n_c = 2 if (m2 // 2) % 16 == 0 else 1
    tc = m2 // n_c
    pad_rows = _round_up(w + 1, 16)
    lp = hw + 2 * pad_rows
    grid = gs_a + n_pair + n_c + n_pair

    w1b = w1.astype(jnp.bfloat16)
    w2r = w2[:, :pr, :pr]
    zz = jnp.zeros_like(w2r)
    w2bd = jnp.concatenate([jnp.concatenate([w2r, zz], axis=2),
                            jnp.concatenate([zz, w2r], axis=2)], axis=1)
    w2p = (w2bd.astype(jnp.bfloat16).reshape(3, 3, cp, cp)
           .transpose(1, 2, 0, 3).reshape(3 * cp, 3 * cp))
    w3a = w3.astype(jnp.bfloat16)
    w3b = jnp.concatenate([jnp.zeros((pr, c4), w3.dtype), w3[:pr]],
                          axis=0).astype(jnp.bfloat16)
    w3p = jnp.concatenate([w3[:pr], w3[:pr]], axis=0)

    g1p = jnp.concatenate([g1[:, :pr], g1[:, :pr]], axis=1)
    b1p = jnp.concatenate([b1[:, :pr], b1[:, :pr]], axis=1)
    g2p = jnp.concatenate([g2[:, :pr], g2[:, :pr]], axis=1)
    b2p = jnp.concatenate([b2[:, :pr], b2[:, :pr]], axis=1)

    col = (jnp.arange(lp, dtype=jnp.int32) - pad_rows) % w
    ml = (col >= 1).astype(jnp.bfloat16).reshape(lp, 1)
    mr = (col <= w - 2).astype(jnp.bfloat16).reshape(lp, 1)
    blk = jnp.arange(cp) // pr
    bd = (blk[:, None] == blk[None, :]).astype(jnp.float32)

    kern = functools.partial(
        _mega_packed_body, gs_a=gs_a, n_pair=n_pair, n_c=n_c, tc=tc,
        hw=hw, width=w, pad_rows=pad_rows, count=count, pr=pr)

    def _x_map(i):
        return (jnp.minimum(i, gs_a - 1), 0)

    def _o_map(i):
        return (jnp.maximum(i - (gs_a + n_pair + n_c), 0), 0)

    return pl.pallas_call(
        kern,
        grid=(grid,),
        in_specs=[pl.BlockSpec((hw, cin_pad), _x_map),
                  pl.BlockSpec((cin_pad, cp), lambda i: (0, 0)),
                  pl.BlockSpec((3 * cp, 3 * cp), lambda i: (0, 0)),
                  pl.BlockSpec((cp, c4), lambda i: (0, 0)),
                  pl.BlockSpec((cp, c4), lambda i: (0, 0)),
                  pl.BlockSpec((cp, c4), lambda i: (0, 0)),
                  pl.BlockSpec((1, cp), lambda i: (0, 0)),
                  pl.BlockSpec((1, cp), lambda i: (0, 0)),
                  pl.BlockSpec((1, cp), lambda i: (0, 0)),
                  pl.BlockSpec((1, cp), lambda i: (0, 0)),
                  pl.BlockSpec((1, c4), lambda i: (0, 0)),
                  pl.BlockSpec((1, c4), lambda i: (0, 0)),
                  pl.BlockSpec((lp, 1), lambda i: (0, 0)),
                  pl.BlockSpec((lp, 1), lambda i: (0, 0)),
                  pl.BlockSpec((cp, cp), lambda i: (0, 0))],
        out_specs=pl.BlockSpec((2 * hw, c4), _o_map),
        out_shape=jax.ShapeDtypeStruct((m, c4), jnp.float32),
        scratch_shapes=[pltpu.VMEM((m, cin_pad), jnp.bfloat16),
                        pltpu.VMEM((m, cp), jnp.bfloat16),
                        pltpu.VMEM((m2, cp), jnp.bfloat16),
                        pltpu.VMEM((1, cp), jnp.float32),
                        pltpu.VMEM((1, cp), jnp.float32),
                        pltpu.VMEM((1, cp), jnp.float32),
                        pltpu.VMEM((1, cp), jnp.float32),
                        pltpu.VMEM((1, cp), jnp.float32),
                        pltpu.VMEM((cp, cp), jnp.float32)],
        compiler_params=pltpu.CompilerParams(
            dimension_semantics=("arbitrary",),
            vmem_limit_bytes=_VMEM_LIMIT),
    )(x2d, w1b, w2p, w3a, w3b, w3p, g1p, b1p, g2p, b2p, g3, b3, ml, mr, bd)


def kernel(x_nhwc, w1, w2, w3, g1, b1, g2, b2, g3, b3):
    n, h, w, cin = x_nhwc.shape
    m = n * h * w
    hw = h * w
    cin_pad = w1.shape[0]

    x2d = x_nhwc.reshape(m, cin)
    if cin_pad != cin:
        x2d = jnp.pad(x2d, ((0, 0), (0, cin_pad - cin)))

    if (n % 2 == 0 and hw % 16 == 0
            and w2.shape[1] == w2.shape[2] == 2 * (cin // 4)):
        out = _kernel_packed(x2d, w1, w2, w3, g1, b1, g2, b2, g3, b3,
                             n, h, w, cin_pad)
        if cin_pad != cin:
            out = out[:, :cin]
        return out.reshape(n, h, w, cin)

    cp = w2.shape[1]
    cout2 = w2.shape[2]
    c4 = w3.shape[1]
    count = float(m)

    tm = hw
    gs_a = m // tm
    n_img = n
    n_c = 2 if (m // 2) % 8 == 0 else 1
    tc = m // n_c
    gs_d = m // tm
    pad_rows = _round_up(w + 1, 16)
    lp = hw + 2 * pad_rows
    grid = gs_a + n_img + n_c + gs_d

    w1b = w1.astype(jnp.bfloat16)
    w2b = (w2.astype(jnp.bfloat16).reshape(3, 3, cp, cout2)
           .transpose(1, 2, 0, 3).reshape(3 * cp, 3 * cout2))

    col = (jnp.arange(lp, dtype=jnp.int32) - pad_rows) % w
    ml = (col >= 1).astype(jnp.bfloat16).reshape(lp, 1)
    mr = (col <= w - 2).astype(jnp.bfloat16).reshape(lp, 1)

    kern = functools.partial(
        _mega_body, gs_a=gs_a, n_img=n_img, n_c=n_c, gs_d=gs_d,
        tm=tm, tc=tc, hw=hw, width=w, pad_rows=pad_rows, count=count)

    def _x_map(i):
        return (jnp.minimum(i, gs_a - 1), 0)

    def _o_map(i):
        return (jnp.maximum(i - (gs_a + n_img + n_c), 0), 0)

    out = pl.pallas_call(
        kern,
        grid=(grid,),
        in_specs=[pl.BlockSpec((tm, cin_pad), _x_map),
                  pl.BlockSpec((cin_pad, cp), lambda i: (0, 0)),
                  pl.BlockSpec((3 * cp, 3 * cout2), lambda i: (0, 0)),
                  pl.BlockSpec((cp, c4), lambda i: (0, 0)),
                  pl.BlockSpec((1, cp), lambda i: (0, 0)),
                  pl.BlockSpec((1, cp), lambda i: (0, 0)),
                  pl.BlockSpec((1, cp), lambda i: (0, 0)),
                  pl.BlockSpec((1, cp), lambda i: (0, 0)),
                  pl.BlockSpec((1, c4), lambda i: (0, 0)),
                  pl.BlockSpec((1, c4), lambda i: (0, 0)),
                  pl.BlockSpec((lp, 1), lambda i: (0, 0)),
                  pl.BlockSpec((lp, 1), lambda i: (0, 0))],
        out_specs=pl.BlockSpec((tm, c4), _o_map),
        out_shape=jax.ShapeDtypeStruct((m, c4), jnp.float32),
        scratch_shapes=[pltpu.VMEM((m, cin_pad), jnp.bfloat16),
                        pltpu.VMEM((m, cp), jnp.bfloat16),
                        pltpu.VMEM((m, cout2), jnp.bfloat16),
                        pltpu.VMEM((1, cp), jnp.float32),
                        pltpu.VMEM((1, cp), jnp.float32),
                        pltpu.VMEM((1, cout2), jnp.float32),
                        pltpu.VMEM((1, cout2), jnp.float32),
                        pltpu.VMEM((1, cout2), jnp.float32),
                        pltpu.VMEM((cout2, cout2), jnp.float32)],
        compiler_params=pltpu.CompilerParams(
            dimension_semantics=("arbitrary",),
            vmem_limit_bytes=_VMEM_LIMIT),
    )(x2d, w1b, w2b, w3, g1, b1, g2, b2, g3, b3, ml, mr)

    if cin_pad != cin:
        out = out[:, :cin]
    return out.reshape(n, h, w, cin)


# speedup vs baseline: 1.0609x; 1.0128x over previous
import functools

import jax
import jax.numpy as jnp
from jax import lax
from jax.experimental import pallas as pl
from jax.experimental.pallas import tpu as pltpu

EPS = 1e-5
_VMEM_LIMIT = 56 * 1024 * 1024


def _round_up(x, m):
    return (x + m - 1) // m * m


def _fold(s, q, g, b, count, eps):
    mean = s * (1.0 / count)
    var = jnp.maximum(q * (1.0 / count) - mean * mean, 0.0)
    inv = lax.rsqrt(var + eps)
    scale = g * inv
    shift = b - mean * scale
    return scale, shift


def _mega_body(x_ref, w1_ref, w2_ref, w3_ref, g1_ref, b1_ref, g2_ref, b2_ref,
               g3_ref, b3_ref, ml_ref, mr_ref, o_ref,
               xb_s, y1_s, y2_s, s1_s, q1_s, s2_s, q2_s, cs_s, gram_s,
               *, gs_a, n_img, n_c, gs_d, tm, tc, hw, width, pad_rows, count):
    i = pl.program_id(0)

    @pl.when(i == 0)
    def _():
        s1_s[...] = jnp.zeros_like(s1_s)
        q1_s[...] = jnp.zeros_like(q1_s)
        s2_s[...] = jnp.zeros_like(s2_s)
        q2_s[...] = jnp.zeros_like(q2_s)
        cs_s[...] = jnp.zeros_like(cs_s)
        gram_s[...] = jnp.zeros_like(gram_s)

    @pl.when(i < gs_a)
    def _():
        x = x_ref[...]
        xb = x.astype(jnp.bfloat16)
        row = pl.multiple_of(i * tm, tm)
        xb_s[pl.ds(row, tm), :] = xb
        y = jnp.dot(xb, w1_ref[...], preferred_element_type=jnp.float32)
        y1_s[pl.ds(row, tm), :] = y.astype(jnp.bfloat16)
        s1_s[...] += jnp.sum(y, axis=0, keepdims=True)
        q1_s[...] += jnp.sum(y * y, axis=0, keepdims=True)

    @pl.when((i >= gs_a) & (i < gs_a + n_img))
    def _():
        img = i - gs_a
        scale, shift = _fold(s1_s[...], q1_s[...], g1_ref[...], b1_ref[...],
                             count, EPS)
        row = pl.multiple_of(img * hw, hw)
        yb = y1_s[pl.ds(row, hw), :]
        a = jnp.maximum(yb.astype(jnp.float32) * scale + shift, 0.0)
        ab = a.astype(jnp.bfloat16)
        cp = ab.shape[1]

        zpad = jnp.zeros((pad_rows, cp), jnp.bfloat16)
        ap = jnp.concatenate([zpad, ab, zpad], axis=0)
        lp = hw + 2 * pad_rows
        zrow = jnp.zeros((1, cp), jnp.bfloat16)
        a_l = jnp.concatenate([zrow, ap[:lp - 1]], axis=0) * ml_ref[...]
        a_r = jnp.concatenate([ap[1:], zrow], axis=0) * mr_ref[...]
        p3 = jnp.concatenate([a_l, ap, a_r], axis=1)

        c_all = jnp.dot(p3, w2_ref[...], preferred_element_type=jnp.float32)
        cout = w2_ref.shape[1] // 3
        acc = (c_all[pad_rows - width: pad_rows - width + hw, 0:cout]
               + c_all[pad_rows: pad_rows + hw, cout:2 * cout]
               + c_all[pad_rows + width: pad_rows + width + hw,
                       2 * cout:3 * cout])

        y2_s[pl.ds(row, hw), :] = acc.astype(jnp.bfloat16)
        s2_s[...] += jnp.sum(acc, axis=0, keepdims=True)
        q2_s[...] += jnp.sum(acc * acc, axis=0, keepdims=True)

    @pl.when((i >= gs_a + n_img) & (i < gs_a + n_img + n_c))
    def _():
        j = i - gs_a - n_img
        scale, shift = _fold(s2_s[...], q2_s[...], g2_ref[...], b2_ref[...],
                             count, EPS)
        row = pl.multiple_of(j * tc, tc)
        yb = y2_s[pl.ds(row, tc), :]
        a2 = jnp.maximum(yb.astype(jnp.float32) * scale + shift, 0.0)
        ab = a2.astype(jnp.bfloat16)
        cs_s[...] += jnp.sum(a2, axis=0, keepdims=True)
        gram_s[...] += lax.dot_general(ab, ab, (((0,), (0,)), ((), ())),
                                       preferred_element_type=jnp.float32)

    @pl.when(i >= gs_a + n_img + n_c)
    def _():
        j = i - gs_a - n_img - n_c
        sc2, sh2 = _fold(s2_s[...], q2_s[...], g2_ref[...], b2_ref[...],
                         count, EPS)
        row = pl.multiple_of(j * tm, tm)
        yb = y2_s[pl.ds(row, tm), :]
        a2 = jnp.maximum(yb.astype(jnp.float32) * sc2 + sh2, 0.0)
        ab = a2.astype(jnp.bfloat16)

        w3 = w3_ref[...]
        s3 = jnp.dot(cs_s[...], w3, preferred_element_type=jnp.float32)
        gw = jnp.dot(gram_s[...], w3, preferred_element_type=jnp.float32)
        q3 = jnp.sum(w3 * gw, axis=0, keepdims=True)
        sc3, sh3 = _fold(s3, q3, g3_ref[...], b3_ref[...], count, EPS)

        y3 = jnp.dot(ab, w3.astype(jnp.bfloat16),
                     preferred_element_type=jnp.float32)
        xres = xb_s[pl.ds(row, tm), :].astype(jnp.float32)
        o_ref[...] = jnp.maximum(y3 * sc3 + sh3 + xres, 0.0)


def _mega_packed_body(x_ref, w1_ref, w2_ref, w3a_ref, w3b_ref, w3p_ref,
                      g1_ref, b1_ref, g2_ref, b2_ref, g3_ref, b3_ref,
                      ml_ref, mr_ref, bd_ref, o_ref,
                      xb_s, y1_s, y2_s, s1_s, q1_s, s2_s, q2_s, cs_s, gram_s,
                      *, gs_a, n_pair, n_c, tc, hw, width, pad_rows, count,
                      pr):
    i = pl.program_id(0)

    @pl.when(i == 0)
    def _():
        s1_s[...] = jnp.zeros_like(s1_s)
        q1_s[...] = jnp.zeros_like(q1_s)
        s2_s[...] = jnp.zeros_like(s2_s)
        q2_s[...] = jnp.zeros_like(q2_s)
        cs_s[...] = jnp.zeros_like(cs_s)
        gram_s[...] = jnp.zeros_like(gram_s)

    def tot(s):
        return s + jnp.concatenate([s[:, pr:], s[:, :pr]], axis=1)

    @pl.when(i < gs_a)
    def _():
        x = x_ref[...]
        xb = x.astype(jnp.bfloat16)
        row = pl.multiple_of(i * 2 * hw, hw)
        xb_s[pl.ds(row, 2 * hw), :] = xb
        y = jnp.dot(xb, w1_ref[...], preferred_element_type=jnp.float32)
        y1_s[pl.ds(row, 2 * hw), :] = y.astype(jnp.bfloat16)
        s1_s[...] += jnp.sum(y, axis=0, keepdims=True)
        q1_s[...] += jnp.sum(y * y, axis=0, keepdims=True)

    @pl.when((i >= gs_a) & (i < gs_a + n_pair))
    def _():
        p = i - gs_a
        scale, shift = _fold(tot(s1_s[...]), tot(q1_s[...]),
                             g1_ref[...], b1_ref[...], count, EPS)
        row = pl.multiple_of(p * hw, hw)
        rowa = pl.multiple_of(p * 2 * hw, hw)
        rowb = pl.multiple_of(p * 2 * hw + hw, hw)
        ypk = jnp.concatenate([y1_s[pl.ds(rowa, hw), 0:pr],
                               y1_s[pl.ds(rowb, hw), 0:pr]], axis=1)
        a = jnp.maximum(ypk.astype(jnp.float32) * scale + shift, 0.0)
        ab = a.astype(jnp.bfloat16)
        cp = ab.shape[1]

        zpad = jnp.zeros((pad_rows, cp), jnp.bfloat16)
        ap = jnp.concatenate([zpad, ab, zpad], axis=0)
        lp = hw + 2 * pad_rows
        zrow = jnp.zeros((1, cp), jnp.bfloat16)
        a_l = jnp.concatenate([zrow, ap[:lp - 1]], axis=0) * ml_ref[...]
        a_r = jnp.concatenate([ap[1:], zrow], axis=0) * mr_ref[...]
        p3 = jnp.concatenate([a_l, ap, a_r], axis=1)

        c_all = jnp.dot(p3, w2_ref[...], preferred_element_type=jnp.float32)
        acc = (c_all[pad_rows - width: pad_rows - width + hw, 0:cp]
               + c_all[pad_rows: pad_rows + hw, cp:2 * cp]
               + c_all[pad_rows + width: pad_rows + width + hw, 2 * cp:3 * cp])

        y2_s[pl.ds(row, hw), :] = acc.astype(jnp.bfloat16)
        s2_s[...] += jnp.sum(acc, axis=0, keepdims=True)
        q2_s[...] += jnp.sum(acc * acc, axis=0, keepdims=True)

    @pl.when((i >= gs_a + n_pair) & (i < gs_a + n_pair + n_c))
    def _():
        j = i - gs_a - n_pair
        scale, shift = _fold(tot(s2_s[...]), tot(q2_s[...]),
                             g2_ref[...], b2_ref[...], count, EPS)
        row = pl.multiple_of(j * tc, tc)
        ypk = y2_s[pl.ds(row, tc), :]
        a2 = jnp.maximum(ypk.astype(jnp.float32) * scale + shift, 0.0)
        ab = a2.astype(jnp.bfloat16)
        cs_s[...] += jnp.sum(a2, axis=0, keepdims=True)
        gram_s[...] += lax.dot_general(ab, ab, (((0,), (0,)), ((), ())),
                                       preferred_element_type=jnp.float32)

    @pl.when(i >= gs_a + n_pair + n_c)
    def _():
        j = i - gs_a - n_pair - n_c
        sc2, sh2 = _fold(tot(s2_s[...]), tot(q2_s[...]),
                         g2_ref[...], b2_ref[...], count, EPS)
        row = pl.multiple_of((j // 2) * hw, hw)
        ypk = y2_s[pl.ds(row, hw), :]
        a2 = jnp.maximum(ypk.astype(jnp.float32) * sc2 + sh2, 0.0)
        ab = a2.astype(jnp.bfloat16)

        w3p = w3p_ref[...]
        gp = gram_s[...] * bd_ref[...]
        s3 = jnp.dot(cs_s[...], w3p, preferred_element_type=jnp.float32)
        gw = jnp.dot(gp, w3p, preferred_element_type=jnp.float32)
        q3 = jnp.sum(w3p * gw, axis=0, keepdims=True)
        sc3, sh3 = _fold(s3, q3, g3_ref[...], b3_ref[...], count, EPS)

        w3h = jnp.where(j % 2 == 0, w3a_ref[...], w3b_ref[...])
        y3 = jnp.dot(ab, w3h, preferred_element_type=jnp.float32)
        xrow = pl.multiple_of(j * hw, hw)
        xx = xb_s[pl.ds(xrow, hw), :].astype(jnp.float32)
        o_ref[...] = jnp.maximum(y3 * sc3 + sh3 + xx, 0.0)


def _kernel_packed(x2d, w1, w2, w3, g1, b1, g2, b2, g3, b3,
                   n, h, w, cin_pad):
    hw = h * w
    m = n * hw
    m2 = m // 2
    cp = w2.shape[1]
    c4 = w3.shape[1]
    pr = cp // 2
    count = float(m)

    gs_a = n // 2
    n_pair = n // 2
    n_c = 2 if (m2 // 2) % 16 == 0 else 1
    tc = m2 // n_c
    pad_rows = _round_up(w + 1, 16)
    lp = hw + 2 * pad_rows
    grid = gs_a + n_pair + n_c + n

    w1b = w1.astype(jnp.bfloat16)
    w2r = w2[:, :pr, :pr]
    zz = jnp.zeros_like(w2r)
    w2bd = jnp.concatenate([jnp.concatenate([w2r, zz], axis=2),
                            jnp.concatenate([zz, w2r], axis=2)], axis=1)
    w2p = (w2bd.astype(jnp.bfloat16).reshape(3, 3, cp, cp)
           .transpose(1, 2, 0, 3).reshape(3 * cp, 3 * cp))
    w3a = w3.astype(jnp.bfloat16)
    w3b = jnp.concatenate([jnp.zeros((pr, c4), w3.dtype), w3[:pr]],
                          axis=0).astype(jnp.bfloat16)
    w3p = jnp.concatenate([w3[:pr], w3[:pr]], axis=0)

    g1p = jnp.concatenate([g1[:, :pr], g1[:, :pr]], axis=1)
    b1p = jnp.concatenate([b1[:, :pr], b1[:, :pr]], axis=1)
    g2p = jnp.concatenate([g2[:, :pr], g2[:, :pr]], axis=1)
    b2p = jnp.concatenate([b2[:, :pr], b2[:, :pr]], axis=1)

    col = (jnp.arange(lp, dtype=jnp.int32) - pad_rows) % w
    ml = (col >= 1).astype(jnp.bfloat16).reshape(lp, 1)
    mr = (col <= w - 2).astype(jnp.bfloat16).reshape(lp, 1)
    blk = jnp.arange(cp) // pr
    bd = (blk[:, None] == blk[None, :]).astype(jnp.float32)

    kern = functools.partial(
        _mega_packed_body, gs_a=gs_a, n_pair=n_pair, n_c=n_c, tc=tc,
        hw=hw, width=w, pad_rows=pad_rows, count=count, pr=pr)

    def _x_map(i):
        return (jnp.minimum(i, gs_a - 1), 0)

    def _o_map(i):
        return (jnp.maximum(i - (gs_a + n_pair + n_c), 0), 0)

    return pl.pallas_call(
        kern,
        grid=(grid,),
        in_specs=[pl.BlockSpec((2 * hw, cin_pad), _x_map),
                  pl.BlockSpec((cin_pad, cp), lambda i: (0, 0)),
                  pl.BlockSpec((3 * cp, 3 * cp), lambda i: (0, 0)),
                  pl.BlockSpec((cp, c4), lambda i: (0, 0)),
                  pl.BlockSpec((cp, c4), lambda i: (0, 0)),
                  pl.BlockSpec((cp, c4), lambda i: (0, 0)),
                  pl.BlockSpec((1, cp), lambda i: (0, 0)),
                  pl.BlockSpec((1, cp), lambda i: (0, 0)),
                  pl.BlockSpec((1, cp), lambda i: (0, 0)),
                  pl.BlockSpec((1, cp), lambda i: (0, 0)),
                  pl.BlockSpec((1, c4), lambda i: (0, 0)),
                  pl.BlockSpec((1, c4), lambda i: (0, 0)),
                  pl.BlockSpec((lp, 1), lambda i: (0, 0)),
                  pl.BlockSpec((lp, 1), lambda i: (0, 0)),
                  pl.BlockSpec((cp, cp), lambda i: (0, 0))],
        out_specs=pl.BlockSpec((hw, c4), _o_map),
        out_shape=jax.ShapeDtypeStruct((m, c4), jnp.float32),
        scratch_shapes=[pltpu.VMEM((m, cin_pad), jnp.bfloat16),
                        pltpu.VMEM((m, cp), jnp.bfloat16),
                        pltpu.VMEM((m2, cp), jnp.bfloat16),
                        pltpu.VMEM((1, cp), jnp.float32),
                        pltpu.VMEM((1, cp), jnp.float32),
                        pltpu.VMEM((1, cp), jnp.float32),
                        pltpu.VMEM((1, cp), jnp.float32),
                        pltpu.VMEM((1, cp), jnp.float32),
                        pltpu.VMEM((cp, cp), jnp.float32)],
        compiler_params=pltpu.CompilerParams(
            dimension_semantics=("arbitrary",),
            vmem_limit_bytes=_VMEM_LIMIT),
    )(x2d, w1b, w2p, w3a, w3b, w3p, g1p, b1p, g2p, b2p, g3, b3, ml, mr, bd)


def kernel(x_nhwc, w1, w2, w3, g1, b1, g2, b2, g3, b3):
    n, h, w, cin = x_nhwc.shape
    m = n * h * w
    hw = h * w
    cin_pad = w1.shape[0]

    x2d = x_nhwc.reshape(m, cin)
    if cin_pad != cin:
        x2d = jnp.pad(x2d, ((0, 0), (0, cin_pad - cin)))

    if (n % 2 == 0 and hw % 16 == 0
            and w2.shape[1] == w2.shape[2] == 2 * (cin // 4)):
        out = _kernel_packed(x2d, w1, w2, w3, g1, b1, g2, b2, g3, b3,
                             n, h, w, cin_pad)
        if cin_pad != cin:
            out = out[:, :cin]
        return out.reshape(n, h, w, cin)

    cp = w2.shape[1]
    cout2 = w2.shape[2]
    c4 = w3.shape[1]
    count = float(m)

    tm = hw
    gs_a = m // tm
    n_img = n
    n_c = 2 if (m // 2) % 8 == 0 else 1
    tc = m // n_c
    gs_d = m // tm
    pad_rows = _round_up(w + 1, 16)
    lp = hw + 2 * pad_rows
    grid = gs_a + n_img + n_c + gs_d

    w1b = w1.astype(jnp.bfloat16)
    w2b = (w2.astype(jnp.bfloat16).reshape(3, 3, cp, cout2)
           .transpose(1, 2, 0, 3).reshape(3 * cp, 3 * cout2))

    col = (jnp.arange(lp, dtype=jnp.int32) - pad_rows) % w
    ml = (col >= 1).astype(jnp.bfloat16).reshape(lp, 1)
    mr = (col <= w - 2).astype(jnp.bfloat16).reshape(lp, 1)

    kern = functools.partial(
        _mega_body, gs_a=gs_a, n_img=n_img, n_c=n_c, gs_d=gs_d,
        tm=tm, tc=tc, hw=hw, width=w, pad_rows=pad_rows, count=count)

    def _x_map(i):
        return (jnp.minimum(i, gs_a - 1), 0)

    def _o_map(i):
        return (jnp.maximum(i - (gs_a + n_img + n_c), 0), 0)

    out = pl.pallas_call(
        kern,
        grid=(grid,),
        in_specs=[pl.BlockSpec((tm, cin_pad), _x_map),
                  pl.BlockSpec((cin_pad, cp), lambda i: (0, 0)),
                  pl.BlockSpec((3 * cp, 3 * cout2), lambda i: (0, 0)),
                  pl.BlockSpec((cp, c4), lambda i: (0, 0)),
                  pl.BlockSpec((1, cp), lambda i: (0, 0)),
                  pl.BlockSpec((1, cp), lambda i: (0, 0)),
                  pl.BlockSpec((1, cp), lambda i: (0, 0)),
                  pl.BlockSpec((1, cp), lambda i: (0, 0)),
                  pl.BlockSpec((1, c4), lambda i: (0, 0)),
                  pl.BlockSpec((1, c4), lambda i: (0, 0)),
                  pl.BlockSpec((lp, 1), lambda i: (0, 0)),
                  pl.BlockSpec((lp, 1), lambda i: (0, 0))],
        out_specs=pl.BlockSpec((tm, c4), _o_map),
        out_shape=jax.ShapeDtypeStruct((m, c4), jnp.float32),
        scratch_shapes=[pltpu.VMEM((m, cin_pad), jnp.bfloat16),
                        pltpu.VMEM((m, cp), jnp.bfloat16),
                        pltpu.VMEM((m, cout2), jnp.bfloat16),
                        pltpu.VMEM((1, cp), jnp.float32),
                        pltpu.VMEM((1, cp), jnp.float32),
                        pltpu.VMEM((1, cout2), jnp.float32),
                        pltpu.VMEM((1, cout2), jnp.float32),
                        pltpu.VMEM((1, cout2), jnp.float32),
                        pltpu.VMEM((cout2, cout2), jnp.float32)],
        compiler_params=pltpu.CompilerParams(
            dimension_semantics=("arbitrary",),
            vmem_limit_bytes=_VMEM_LIMIT),
    )(x2d, w1b, w2b, w3, g1, b1, g2, b2, g3, b3, ml, mr)

    if cin_pad != cin:
        out = out[:, :cin]
    return out.reshape(n, h, w, cin)
